# baseline (speedup 1.0000x reference)
import functools

import jax
import jax.numpy as jnp
from jax import lax
from jax.experimental import pallas as pl
from jax.experimental.pallas import tpu as pltpu
from jax.experimental.pallas import tpu_sc as plsc

B = 16384
D = 64
NC = 2
NS = 16
NW = NC * NS
BPW = B // NW
VL = 16

_mesh = plsc.VectorSubcoreMesh(core_axis_name="c", subcore_axis_name="s")


@functools.partial(
    pl.kernel,
    mesh=_mesh,
    out_type=(
        jax.ShapeDtypeStruct((B, D), jnp.float32),
        jax.ShapeDtypeStruct((B, D), jnp.float32),
        jax.ShapeDtypeStruct((B, D), jnp.float32),
    ),
    scratch_types=(
        pltpu.VMEM((BPW,), jnp.int32),
        pltpu.VMEM((BPW,), jnp.int32),
        pltpu.VMEM((BPW,), jnp.int32),
        pltpu.SemaphoreType.DMA,
        pltpu.SemaphoreType.DMA,
        pltpu.SemaphoreType.DMA,
    ),
)
def _gather3(uid, iid, cid, ut, it, ct, ou, oi, oc,
             is0, is1, is2, sg0, sg1, sg2):
    wid = lax.axis_index("s") * NC + lax.axis_index("c")
    base = wid * BPW
    tabs = ((uid, ut, ou, sg0, is0), (iid, it, oi, sg1, is1),
            (cid, ct, oc, sg2, is2))

    for idx_hbm, _, _, _, idx_v in tabs:
        pltpu.sync_copy(idx_hbm.at[pl.ds(base, BPW)], idx_v)

    for _, tab, out, sg, idx_v in tabs:
        def body(s, carry, tab=tab, out=out, sg=sg, idx_v=idx_v):
            rbase = s * VL
            vec = idx_v[pl.ds(rbase, VL)]
            for u in range(VL):
                pltpu.async_copy(tab.at[vec[u]], out.at[base + rbase + u], sg)
            return carry
        lax.fori_loop(0, BPW // VL, body, 0)

    for _, tab, out, sg, _ in tabs:
        pltpu.make_async_copy(
            tab.at[pl.ds(0, BPW)], out.at[pl.ds(base, BPW)], sg).wait()


def kernel(user_id, item_id, category_id, user_table, item_table, cat_table):
    uid = user_id.reshape(B)
    iid = item_id.reshape(B)
    cid = category_id.reshape(B)
    return _gather3(uid, iid, cid, user_table, item_table, cat_table)


# speedup vs baseline: 1.2377x; 1.2377x over previous
import functools

import jax
import jax.numpy as jnp
from jax import lax
from jax.experimental import pallas as pl
from jax.experimental.pallas import tpu as pltpu
from jax.experimental.pallas import tpu_sc as plsc

B = 16384
D = 64
NC = 2
NS = 16
NW = NC * NS
BPW = B // NW
CH = 128
NCH = BPW // CH
VL = 16
NU = 3 * NCH

_mesh = plsc.VectorSubcoreMesh(core_axis_name="c", subcore_axis_name="s")


@functools.partial(
    pl.kernel,
    mesh=_mesh,
    out_type=(
        jax.ShapeDtypeStruct((B, D), jnp.float32),
        jax.ShapeDtypeStruct((B, D), jnp.float32),
        jax.ShapeDtypeStruct((B, D), jnp.float32),
    ),
    scratch_types=(
        pltpu.VMEM((NCH, CH), jnp.int32),
        pltpu.VMEM((NCH, CH), jnp.int32),
        pltpu.VMEM((NCH, CH), jnp.int32),
        pltpu.VMEM((NCH, CH), jnp.int32),
        pltpu.VMEM((NCH, CH), jnp.int32),
        pltpu.VMEM((NCH, CH), jnp.int32),
        pltpu.VMEM((2, CH, 2 * D), jnp.float32),
        pltpu.VMEM((2, CH, D), jnp.float32),
        pltpu.SemaphoreType.DMA,
        pltpu.SemaphoreType.DMA,
        pltpu.SemaphoreType.DMA,
        pltpu.SemaphoreType.DMA,
    ),
)
def _gather3(uid, iid, cid, ut2, it2, ct2, ou, oi, oc,
             l0, l1, l2, p0, p1, p2, prow, outb, g0, g1, w0, w1):
    wid = lax.axis_index("s") * NC + lax.axis_index("c")
    base = wid * BPW
    srcs = (uid, iid, cid)
    tabs = (ut2, it2, ct2)
    outs = (ou, oi, oc)
    lands = (l0, l1, l2)
    pidxs = (p0, p1, p2)
    gsems = (g0, g1)
    wsems = (w0, w1)

    for t in range(3):
        pltpu.sync_copy(srcs[t].at[pl.ds(wid * NCH, NCH)], lands[t])
        for s in range(BPW // VL):
            v = lands[t][s // (CH // VL), pl.ds((s % (CH // VL)) * VL, VL)]
            pidxs[t][s // (CH // VL), pl.ds((s % (CH // VL)) * VL, VL)] = (
                v >> 1)

    units = [(t, c) for t in range(3) for c in range(NCH)]
    gh = [None] * NU
    wh = [None] * NU

    def fire(u):
        t, c = units[u]
        gh[u] = pltpu.async_copy(
            tabs[t].at[pidxs[t].at[c]], prow.at[u % 2], gsems[u % 2])

    fire(0)
    for u, (t, c) in enumerate(units):
        slot = u % 2
        if u + 1 < NU:
            fire(u + 1)
        gh[u].wait()
        if u >= 2:
            wh[u - 2].wait()

        def cgroup(g, carry, t=t, c=c, slot=slot):
            mv = lands[t][c, pl.ds(g * VL, VL)]
            for u in range(VL):
                oddf = jnp.zeros((VL,), jnp.float32) + (
                    (mv[u] & 1).astype(jnp.float32))
                r = g * VL + u
                for j in range(D // VL):
                    a = prow[slot, r, pl.ds(j * VL, VL)]
                    b = prow[slot, r, pl.ds(D + j * VL, VL)]
                    outb[slot, r, pl.ds(j * VL, VL)] = a + (b - a) * oddf
            return carry

        lax.fori_loop(0, CH // VL, cgroup, 0)
        wh[u] = pltpu.async_copy(
            outb.at[slot], outs[t].at[pl.ds(base + c * CH, CH)], wsems[slot])

    wh[NU - 2].wait()
    wh[NU - 1].wait()


def kernel(user_id, item_id, category_id, user_table, item_table, cat_table):
    uid = user_id.reshape(NW * NCH, CH)
    iid = item_id.reshape(NW * NCH, CH)
    cid = category_id.reshape(NW * NCH, CH)
    ut2 = user_table.reshape(-1, 2 * D)
    it2 = item_table.reshape(-1, 2 * D)
    ct2 = cat_table.reshape(-1, 2 * D)
    return _gather3(uid, iid, cid, ut2, it2, ct2)


# speedup vs baseline: 1.9409x; 1.5682x over previous
import functools

import jax
import jax.numpy as jnp
from jax import lax
from jax.experimental import pallas as pl
from jax.experimental.pallas import tpu as pltpu
from jax.experimental.pallas import tpu_sc as plsc

B = 16384
D = 64
NC = 2
NS = 16
NW = NC * NS
BPW = B // NW
CH = 64
NCH = BPW // CH
VL = 16

_mesh = plsc.VectorSubcoreMesh(core_axis_name="c", subcore_axis_name="s")


@functools.partial(
    pl.kernel,
    mesh=_mesh,
    out_type=(
        jax.ShapeDtypeStruct((B, D), jnp.float32),
        jax.ShapeDtypeStruct((B, D), jnp.float32),
        jax.ShapeDtypeStruct((B, D), jnp.float32),
    ),
    scratch_types=(
        pltpu.VMEM((4, 128), jnp.int32),
        pltpu.VMEM((4, 128), jnp.int32),
        pltpu.VMEM((4, 128), jnp.int32),
        pltpu.VMEM((2, CH, D), jnp.float32),
        pltpu.SemaphoreType.DMA,
        pltpu.SemaphoreType.DMA,
        pltpu.SemaphoreType.DMA,
        pltpu.SemaphoreType.DMA,
    ),
)
def _gather3(uid, iid, cid, ut, it, ct, ou, oi, oc,
             l0, l1, l2, buf, g0, g1, w0, w1):
    wid = lax.axis_index("s") * NC + lax.axis_index("c")
    base = wid * BPW
    srcs = (uid, iid, cid)
    tabs = (ut, it, ct)
    outs = (ou, oi, oc)
    lands = (l0, l1, l2)
    gsems = (g0, g1)
    wsems = (w0, w1)

    for t in range(3):
        pltpu.sync_copy(srcs[t].at[pl.ds(wid * 4, 4)], lands[t])

    def fire(t, c, slot):
        hs = []
        for g in range(CH // VL):
            mv = lands[t][(c * CH + g * VL) // 128,
                          pl.ds((c * CH + g * VL) % 128, VL)]
            for u in range(VL):
                r = g * VL + u
                hs.append(pltpu.async_copy(
                    tabs[t].at[pl.ds(mv[u], 1)],
                    buf.at[slot].at[pl.ds(r, 1)],
                    gsems[slot]))
        return hs

    for t in range(3):
        def body(i, carry, t=t):
            gh = [None, None]
            wh = [None, None]
            for b in range(2):
                gh[b] = fire(t, 2 * i + b, b)
            for b in range(2):
                c = 2 * i + b
                for h in gh[b]:
                    h.wait()
                wh[b] = pltpu.async_copy(
                    buf.at[b], outs[t].at[pl.ds(base + c * CH, CH)],
                    wsems[b])
            wh[0].wait()
            wh[1].wait()
            return carry

        lax.fori_loop(0, NCH // 2, body, 0)


def kernel(user_id, item_id, category_id, user_table, item_table, cat_table):
    uid = user_id.reshape(NW * 4, 128)
    iid = item_id.reshape(NW * 4, 128)
    cid = category_id.reshape(NW * 4, 128)
    return _gather3(uid, iid, cid, user_table, item_table, cat_table)
